# speedup vs baseline: 19.4620x; 19.4620x over previous
import functools

import jax
import jax.numpy as jnp
from jax.experimental import pallas as pl
from jax.experimental.pallas import tpu as pltpu


def _shift_fwd(x, k, mask, pq):
    if k:
        x = jnp.concatenate(
            [jnp.zeros((k, x.shape[1]), x.dtype), x[: pq - k, :]], axis=0)
    if mask is not None:
        x = jnp.where(mask, 0.0, x)
    return x


def _shift_bwd(x, k, mask, pq):
    if k:
        x = jnp.concatenate(
            [x[k:, :], jnp.zeros((k, x.shape[1]), x.dtype)], axis=0)
    if mask is not None:
        x = jnp.where(mask, 0.0, x)
    return x


def _lrp_kernel(a_ref, r_ref,
                w1f_ref, w1p_ref, w1pb_ref,
                w2f_ref, w2p_ref, w2pb_ref,
                w3f_ref, w3p_ref, w3pb_ref,
                wdf_ref, wdp_ref, wdpb_ref,
                s1_ref, b1_ref, s2_ref, b2_ref,
                s3_ref, b3_ref, sd_ref, bd_ref,
                out_ref, *, P, Q, width, eps):
    pq = P * Q
    f32 = jnp.float32
    A = a_ref[0]
    R = r_ref[0]

    def mm(x, w):
        return jax.lax.dot_general(
            x, w, (((1,), (0,)), ((), ())), preferred_element_type=f32)

    h1 = jnp.maximum(mm(A, w1f_ref[...]) * s1_ref[...] + b1_ref[...], 0.0)
    z1 = mm(A, w1p_ref[...])

    qidx = jax.lax.broadcasted_iota(jnp.int32, (pq, width), 0) % Q
    mask_q0 = qidx == 0
    mask_qL = qidx == Q - 1

    tap_geo = [(1, -1), (0, 0), (1, 0)]

    h2acc = jnp.zeros((pq, width), f32)
    z2acc = jnp.zeros((pq, width), f32)
    for kh in range(3):
        sh, dp = tap_geo[kh]
        for kw in range(3):
            sw, dq = tap_geo[kw]
            t = kh * 3 + kw
            plane = h1[(2 * sh + sw) * pq:(2 * sh + sw + 1) * pq, :]
            k = -(dp * Q + dq)
            tap = _shift_fwd(plane, k, mask_q0 if dq else None, pq)
            h2acc += mm(tap, w2f_ref[t * width:(t + 1) * width, :])
            z2acc += mm(tap, w2p_ref[t * width:(t + 1) * width, :])
    h2 = jnp.maximum(h2acc * s2_ref[...] + b2_ref[...], 0.0)

    mstream = mm(h2, w3f_ref[...]) * s3_ref[...] + b3_ref[...]
    z3 = mm(h2, w3p_ref[...])
    A00 = A[:pq, :]
    short = mm(A00, wdf_ref[...]) * sd_ref[...] + bd_ref[...]
    zd = mm(A00, wdp_ref[...])
    am = jnp.abs(mstream)
    ratio = am / (am + jnp.abs(short))
    r_main = ratio * R
    r_short = (1.0 - ratio) * R

    s3v = r_main / (z3 + eps)
    r2 = h2 * mm(s3v, w3pb_ref[...])

    s2v = r2 / (z2acc + eps)
    c2_planes = [jnp.zeros((pq, width), f32) for _ in range(4)]
    for kh in range(3):
        sh, dp = tap_geo[kh]
        for kw in range(3):
            sw, dq = tap_geo[kw]
            t = kh * 3 + kw
            g = mm(s2v, w2pb_ref[t * width:(t + 1) * width, :])
            k = -(dp * Q + dq)
            c2_planes[2 * sh + sw] += _shift_bwd(
                g, k, mask_qL if dq else None, pq)
    c2 = jnp.concatenate(c2_planes, axis=0)
    r1 = h1 * c2

    s1v = r1 / (z1 + eps)
    c1 = mm(s1v, w1pb_ref[...])
    sdv = r_short / (zd + eps)
    cd = mm(sdv, wdpb_ref[...])

    out_ref[0, :pq, :] = A00 * (c1[:pq, :] + cd)
    out_ref[0, pq:, :] = A[pq:, :] * c1[pq:, :]


def kernel(a, r, w1, w2, w3, wd,
           bn1_g, bn1_b, bn1_mu, bn1_v,
           bn2_g, bn2_b, bn2_mu, bn2_v,
           bn3_g, bn3_b, bn3_mu, bn3_v,
           bnd_g, bnd_b, bnd_mu, bnd_v):
    eps = 1e-5
    bn_eps = 1e-5
    n, cin, h, w = a.shape
    cout = r.shape[1]
    width = w1.shape[0]
    P, Q = h // 2, w // 2
    pq = P * Q

    A = a.reshape(n, cin, P, 2, Q, 2).transpose(0, 3, 5, 2, 4, 1)
    A = A.reshape(n, 4 * pq, cin)
    R = r.transpose(0, 2, 3, 1).reshape(n, pq, cout)

    w1f = w1.reshape(width, cin).T
    w1p = jnp.maximum(w1f, 0.0)
    w1pb = w1p.T
    w3f = w3.reshape(cout, width).T
    w3p = jnp.maximum(w3f, 0.0)
    w3pb = w3p.T
    wdf = wd.reshape(cout, cin).T
    wdp = jnp.maximum(wdf, 0.0)
    wdpb = wdp.T
    w2f = w2.transpose(2, 3, 1, 0).reshape(9 * width, width)
    w2p = jnp.maximum(w2f, 0.0)
    w2pb = jnp.maximum(w2, 0.0).transpose(2, 3, 0, 1).reshape(9 * width, width)

    def bn_fold(g, b, mu, v):
        s = g / jnp.sqrt(v + bn_eps)
        return s.reshape(1, -1), (b - mu * s).reshape(1, -1)

    s1, b1 = bn_fold(bn1_g, bn1_b, bn1_mu, bn1_v)
    s2, b2 = bn_fold(bn2_g, bn2_b, bn2_mu, bn2_v)
    s3, b3 = bn_fold(bn3_g, bn3_b, bn3_mu, bn3_v)
    sd, bd = bn_fold(bnd_g, bnd_b, bnd_mu, bnd_v)

    def fixed(x):
        return pl.BlockSpec(x.shape, lambda i: (0,) * x.ndim)

    weights = [w1f, w1p, w1pb, w2f, w2p, w2pb, w3f, w3p, w3pb,
               wdf, wdp, wdpb, s1, b1, s2, b2, s3, b3, sd, bd]

    out = pl.pallas_call(
        functools.partial(_lrp_kernel, P=P, Q=Q, width=width, eps=eps),
        out_shape=jax.ShapeDtypeStruct((n, 4 * pq, cin), jnp.float32),
        grid=(n,),
        in_specs=[pl.BlockSpec((1, 4 * pq, cin), lambda i: (i, 0, 0)),
                  pl.BlockSpec((1, pq, cout), lambda i: (i, 0, 0))]
                 + [fixed(x) for x in weights],
        out_specs=pl.BlockSpec((1, 4 * pq, cin), lambda i: (i, 0, 0)),
        compiler_params=pltpu.CompilerParams(
            dimension_semantics=("parallel",),
            vmem_limit_bytes=100 * 1024 * 1024),
    )(A, R, *weights)

    out = out.reshape(n, 2, 2, P, Q, cin).transpose(0, 5, 3, 1, 4, 2)
    return out.reshape(n, cin, h, w)


# speedup vs baseline: 19.4903x; 1.0015x over previous
import functools

import jax
import jax.numpy as jnp
from jax.experimental import pallas as pl
from jax.experimental.pallas import tpu as pltpu


def _lane_shift(x, d, s):
    if d == 0:
        return x
    rows = x.shape[0]
    if d > 0:
        return jnp.concatenate(
            [x[:, d:], jnp.zeros((rows, d), x.dtype)], axis=1)
    return jnp.concatenate(
        [jnp.zeros((rows, -d), x.dtype), x[:, :s + d]], axis=1)


def _lrp_kernel(a_ref, re_ref,
                w1m_ref, w1mp_ref, w1bp_ref,
                w2m_ref, w2mp_ref, w2bp_ref,
                w3m_ref, w3mp_ref, w3bp_ref,
                wdm_ref, wdmp_ref, wdbp_ref,
                s1_ref, b1_ref, s2_ref, b2_ref,
                s3_ref, b3_ref, sd_ref, bd_ref,
                out_ref, *, H, W, width, cout, eps):
    S = H * W
    f32 = jnp.float32
    A = a_ref[0]
    Remb = re_ref[0]

    def mm(w, x):
        return jax.lax.dot_general(
            w, x, (((1,), (0,)), ((), ())), preferred_element_type=f32)

    h1 = jnp.maximum(mm(w1m_ref[...], A) * s1_ref[...] + b1_ref[...], 0.0)
    z1 = mm(w1mp_ref[...], A)

    lcol = jax.lax.broadcasted_iota(jnp.int32, (width, S), 1)
    wmod = lcol % W
    mask_w0 = wmod == 0
    mask_wL = wmod == W - 1

    z2w = jnp.zeros((width, S), f32)
    z2p = jnp.zeros((width, S), f32)
    for kh in range(3):
        for kw in range(3):
            t = kh * 3 + kw
            d = (kh - 1) * W + (kw - 1)
            tap = _lane_shift(h1, d, S)
            if kw == 0:
                tap = jnp.where(mask_w0, 0.0, tap)
            elif kw == 2:
                tap = jnp.where(mask_wL, 0.0, tap)
            z2w += mm(w2m_ref[t * width:(t + 1) * width, :], tap)
            z2p += mm(w2mp_ref[t * width:(t + 1) * width, :], tap)
    h2 = jnp.maximum(z2w * s2_ref[...] + b2_ref[...], 0.0)

    mstream = mm(w3m_ref[...], h2) * s3_ref[...] + b3_ref[...]
    z3 = mm(w3mp_ref[...], h2)
    short = mm(wdm_ref[...], A) * sd_ref[...] + bd_ref[...]
    zd = mm(wdmp_ref[...], A)
    ccol = jax.lax.broadcasted_iota(jnp.int32, (cout, S), 1)
    odd = (((ccol % (2 * W)) >= W) | ((ccol % 2) == 1)).astype(f32)
    am = jnp.abs(mstream)
    ratio = am / (am + jnp.abs(short) + odd)
    r_main = ratio * Remb
    r_short = (1.0 - ratio) * Remb

    s3v = r_main / (z3 + eps)
    r2 = h2 * mm(w3bp_ref[...], s3v)

    s2v = r2 / (z2p + eps)
    c2 = jnp.zeros((width, S), f32)
    for kh in range(3):
        for kw in range(3):
            t = kh * 3 + kw
            d = (kh - 1) * W + (kw - 1)
            g = mm(w2bp_ref[t * width:(t + 1) * width, :], s2v)
            if kw == 0:
                g = jnp.where(mask_w0, 0.0, g)
            elif kw == 2:
                g = jnp.where(mask_wL, 0.0, g)
            c2 += _lane_shift(g, -d, S)
    r1 = h1 * c2

    s1v = r1 / (z1 + eps)
    c1 = mm(w1bp_ref[...], s1v)
    sdv = r_short / (zd + eps)
    cd = mm(wdbp_ref[...], sdv)

    out_ref[0] = A * (c1 + cd)


def kernel(a, r, w1, w2, w3, wd,
           bn1_g, bn1_b, bn1_mu, bn1_v,
           bn2_g, bn2_b, bn2_mu, bn2_v,
           bn3_g, bn3_b, bn3_mu, bn3_v,
           bnd_g, bnd_b, bnd_mu, bnd_v):
    eps = 1e-5
    bn_eps = 1e-5
    n, cin, h, w = a.shape
    cout = r.shape[1]
    width = w1.shape[0]
    S = h * w

    A = a.reshape(n, cin, S)
    Remb = jnp.zeros((n, cout, h, w), r.dtype).at[:, :, ::2, ::2].set(r)
    Remb = Remb.reshape(n, cout, S)

    w1m = w1.reshape(width, cin)
    w1mp = jnp.maximum(w1m, 0.0)
    w1bp = w1mp.T
    w3m = w3.reshape(cout, width)
    w3mp = jnp.maximum(w3m, 0.0)
    w3bp = w3mp.T
    wdm = wd.reshape(cout, cin)
    wdmp = jnp.maximum(wdm, 0.0)
    wdbp = wdmp.T
    w2m = w2.transpose(2, 3, 0, 1).reshape(9 * width, width)
    w2mp = jnp.maximum(w2m, 0.0)
    w2bp = jnp.maximum(w2, 0.0).transpose(2, 3, 1, 0).reshape(9 * width, width)

    def bn_fold(g, b, mu, v):
        s = g / jnp.sqrt(v + bn_eps)
        return s.reshape(-1, 1), (b - mu * s).reshape(-1, 1)

    s1, b1 = bn_fold(bn1_g, bn1_b, bn1_mu, bn1_v)
    s2, b2 = bn_fold(bn2_g, bn2_b, bn2_mu, bn2_v)
    s3, b3 = bn_fold(bn3_g, bn3_b, bn3_mu, bn3_v)
    sd, bd = bn_fold(bnd_g, bnd_b, bnd_mu, bnd_v)

    def fixed(x):
        return pl.BlockSpec(x.shape, lambda i: (0,) * x.ndim)

    weights = [w1m, w1mp, w1bp, w2m, w2mp, w2bp, w3m, w3mp, w3bp,
               wdm, wdmp, wdbp, s1, b1, s2, b2, s3, b3, sd, bd]

    out = pl.pallas_call(
        functools.partial(_lrp_kernel, H=h, W=w, width=width, cout=cout,
                          eps=eps),
        out_shape=jax.ShapeDtypeStruct((n, cin, S), jnp.float32),
        grid=(n,),
        in_specs=[pl.BlockSpec((1, cin, S), lambda i: (i, 0, 0)),
                  pl.BlockSpec((1, cout, S), lambda i: (i, 0, 0))]
                 + [fixed(x) for x in weights],
        out_specs=pl.BlockSpec((1, cin, S), lambda i: (i, 0, 0)),
        compiler_params=pltpu.CompilerParams(
            dimension_semantics=("parallel",),
            vmem_limit_bytes=100 * 1024 * 1024),
    )(A, Remb, *weights)

    return out.reshape(n, cin, h, w)


# speedup vs baseline: 23.9610x; 1.2294x over previous
import functools

import jax
import jax.numpy as jnp
from jax.experimental import pallas as pl
from jax.experimental.pallas import tpu as pltpu


def _lane_shift(x, d, s):
    if d == 0:
        return x
    rows = x.shape[0]
    if d > 0:
        return jnp.concatenate(
            [x[:, d:], jnp.zeros((rows, d), x.dtype)], axis=1)
    return jnp.concatenate(
        [jnp.zeros((rows, -d), x.dtype), x[:, :s + d]], axis=1)


def _tap_matrix(x, mask_w0, mask_wL, W, S, sign):
    xm = {0: jnp.where(mask_wL if sign > 0 else mask_w0, 0.0, x),
          1: x,
          2: jnp.where(mask_w0 if sign > 0 else mask_wL, 0.0, x)}
    taps = []
    for kh in range(3):
        for kw in range(3):
            d = sign * ((kh - 1) * W + (kw - 1))
            taps.append(_lane_shift(xm[kw], d, S))
    return jnp.concatenate(taps, axis=0)


def _lrp_kernel(a_ref, re_ref,
                wfs_ref, w2s_ref, w2bk_ref, w3s_ref, w3bp_ref, wc_ref,
                s1_ref, b1_ref, s2_ref, b2_ref,
                s3_ref, b3_ref, sd_ref, bd_ref,
                out_ref, *, H, W, width, cout, eps):
    S = H * W
    A = a_ref[0]
    Remb = re_ref[0]

    def mm(w, x):
        return jax.lax.dot_general(
            w, x, (((1,), (0,)), ((), ())),
            preferred_element_type=jnp.float32)

    fs = mm(wfs_ref[...], A)
    h1 = jnp.maximum(fs[:width] * s1_ref[...] + b1_ref[...], 0.0)
    z1 = fs[width:2 * width]
    short = fs[2 * width:2 * width + cout] * sd_ref[...] + bd_ref[...]
    zd = fs[2 * width + cout:]

    wmod = jax.lax.broadcasted_iota(jnp.int32, (1, S), 1) % W
    mask_w0 = wmod == 0
    mask_wL = wmod == W - 1

    taps = _tap_matrix(h1, mask_w0, mask_wL, W, S, 1)
    z2 = mm(w2s_ref[...], taps)
    h2 = jnp.maximum(z2[:width] * s2_ref[...] + b2_ref[...], 0.0)
    z2p = z2[width:]

    mz3 = mm(w3s_ref[...], h2)
    mstream = mz3[:cout] * s3_ref[...] + b3_ref[...]
    z3 = mz3[cout:]
    am = jnp.abs(mstream)
    ratio = am / (am + jnp.abs(short) + 1e-30)
    r_main = ratio * Remb
    r_short = Remb - r_main

    s3v = r_main / (z3 + eps)
    r2 = h2 * mm(w3bp_ref[...], s3v)

    s2v = r2 / (z2p + eps)
    btaps = _tap_matrix(s2v, mask_w0, mask_wL, W, S, -1)
    c2 = mm(w2bk_ref[...], btaps)
    r1 = h1 * c2

    s1v = r1 / (z1 + eps)
    sdv = r_short / (zd + eps)
    sv = jnp.concatenate([s1v, sdv], axis=0)
    out_ref[0] = A * mm(wc_ref[...], sv)


def kernel(a, r, w1, w2, w3, wd,
           bn1_g, bn1_b, bn1_mu, bn1_v,
           bn2_g, bn2_b, bn2_mu, bn2_v,
           bn3_g, bn3_b, bn3_mu, bn3_v,
           bnd_g, bnd_b, bnd_mu, bnd_v):
    eps = 1e-5
    bn_eps = 1e-5
    n, cin, h, w = a.shape
    cout = r.shape[1]
    width = w1.shape[0]
    S = h * w

    A = a.reshape(n, cin, S)
    Remb = jnp.pad(r.reshape(n, cout, h // 2, 1, w // 2, 1),
                   ((0, 0), (0, 0), (0, 0), (0, 1), (0, 0), (0, 1)))
    Remb = Remb.reshape(n, cout, S)

    w1m = w1.reshape(width, cin)
    w1mp = jnp.maximum(w1m, 0.0)
    wdm = wd.reshape(cout, cin)
    wdmp = jnp.maximum(wdm, 0.0)
    wfs = jnp.concatenate([w1m, w1mp, wdm, wdmp], axis=0)
    w3m = w3.reshape(cout, width)
    w3mp = jnp.maximum(w3m, 0.0)
    w3s = jnp.concatenate([w3m, w3mp], axis=0)
    w3bp = w3mp.T
    w2k = w2.transpose(0, 2, 3, 1).reshape(width, 9 * width)
    w2s = jnp.concatenate([w2k, jnp.maximum(w2k, 0.0)], axis=0)
    w2bk = jnp.maximum(w2, 0.0).transpose(1, 2, 3, 0).reshape(width, 9 * width)
    wc = jnp.concatenate([w1mp.T, wdmp.T], axis=1)

    def bn_fold(g, b, mu, v):
        s = g / jnp.sqrt(v + bn_eps)
        return s.reshape(-1, 1), (b - mu * s).reshape(-1, 1)

    s1, b1 = bn_fold(bn1_g, bn1_b, bn1_mu, bn1_v)
    s2, b2 = bn_fold(bn2_g, bn2_b, bn2_mu, bn2_v)
    s3, b3 = bn_fold(bn3_g, bn3_b, bn3_mu, bn3_v)
    sd, bd = bn_fold(bnd_g, bnd_b, bnd_mu, bnd_v)

    def fixed(x):
        return pl.BlockSpec(x.shape, lambda i: (0,) * x.ndim)

    weights = [wfs, w2s, w2bk, w3s, w3bp, wc,
               s1, b1, s2, b2, s3, b3, sd, bd]

    out = pl.pallas_call(
        functools.partial(_lrp_kernel, H=h, W=w, width=width, cout=cout,
                          eps=eps),
        out_shape=jax.ShapeDtypeStruct((n, cin, S), jnp.float32),
        grid=(n,),
        in_specs=[pl.BlockSpec((1, cin, S), lambda i: (i, 0, 0)),
                  pl.BlockSpec((1, cout, S), lambda i: (i, 0, 0))]
                 + [fixed(x) for x in weights],
        out_specs=pl.BlockSpec((1, cin, S), lambda i: (i, 0, 0)),
        compiler_params=pltpu.CompilerParams(
            dimension_semantics=("parallel",),
            vmem_limit_bytes=100 * 1024 * 1024),
    )(A, Remb, *weights)

    return out.reshape(n, cin, h, w)
